# Initial kernel scaffold; baseline (speedup 1.0000x reference)
import jax
import jax.numpy as jnp
from jax.experimental import pallas as pl


def kernel(x, edge_index, W1, b1, W2, b2):
    raise NotImplementedError("write your pallas kernel here")


# speedup vs baseline: 4.7691x; 4.7691x over previous
import functools

import jax
import jax.numpy as jnp
from jax import lax
from jax.experimental import pallas as pl
from jax.experimental.pallas import tpu as pltpu
from jax.experimental.pallas import tpu_sc as plsc

N_NODES = 10000
N_EDGES = 320000
DIM = 128
NC = 2
NS = 16
NW = NC * NS
L = 16
EPW = N_EDGES // NW
CHUNK = 80
NCHUNK = EPW // CHUNK
NF = DIM // L


def _proj_body(x_ref, wa_ref, wb_ref, b1_ref, u_ref, v_ref):
    xv = x_ref[...]
    u_ref[...] = jnp.dot(xv, wa_ref[...], preferred_element_type=jnp.float32)
    v_ref[...] = (
        jnp.dot(xv, wb_ref[...], preferred_element_type=jnp.float32) + b1_ref[...]
    )


def _node_projections(x, W1, b1):
    wa = W1[:, :DIM].T
    wb = W1[:, DIM:].T
    return pl.pallas_call(
        _proj_body,
        out_shape=[
            jax.ShapeDtypeStruct((N_NODES, DIM), jnp.float32),
            jax.ShapeDtypeStruct((N_NODES, DIM), jnp.float32),
        ],
    )(x, wa, wb, b1.reshape(1, DIM))


def _edge_body(u_hbm, v_hbm, row_hbm, col_hbm, w2_hbm, b2_hbm, out_hbm,
               idx_r, idx_c, urows, vrows, logv, w2v, b2v, sem):
    cid = lax.axis_index("c")
    sid = lax.axis_index("s")
    wid = sid * NC + cid
    base = wid * EPW

    pltpu.sync_copy(row_hbm.at[wid], idx_r)
    pltpu.sync_copy(col_hbm.at[wid], idx_c)
    pltpu.sync_copy(w2_hbm, w2v)
    pltpu.sync_copy(b2_hbm, b2v)
    b2reg = b2v[...]
    w2regs = [w2v[pl.ds(b * L, L)] for b in range(NF)]
    lane = lax.iota(jnp.int32, L)

    def issue(cc, s):
        pltpu.async_copy(u_hbm.at[idx_r.at[cc]], urows.at[s], sem.at[s])
        pltpu.async_copy(v_hbm.at[idx_c.at[cc]], vrows.at[s], sem.at[s])

    def wait(cc, s):
        pltpu.make_async_copy(u_hbm.at[idx_r.at[cc]], urows.at[s], sem.at[s]).wait()
        pltpu.make_async_copy(v_hbm.at[idx_c.at[cc]], vrows.at[s], sem.at[s]).wait()

    def compute(cc, s):
        ur = urows.at[s]
        vr = vrows.at[s]

        def group_body(g, carry):
            merged = jnp.zeros((L,), jnp.float32)
            for j in range(L):
                e = g * L + j
                acc = jnp.maximum(ur[e, pl.ds(0, L)] + vr[e, pl.ds(0, L)], 0.0)
                acc = acc * w2regs[0]
                for b in range(1, NF):
                    h = jnp.maximum(
                        ur[e, pl.ds(b * L, L)] + vr[e, pl.ds(b * L, L)], 0.0
                    )
                    acc = acc + h * w2regs[b]
                merged = jnp.where(lane == j, jnp.sum(acc), merged)
            z = merged + b2reg
            logv[pl.ds(g * L, L)] = 1.0 / (1.0 + jnp.exp(-z))
            return carry

        lax.fori_loop(0, CHUNK // L, group_body, 0, unroll=False)
        pltpu.sync_copy(logv, out_hbm.at[pl.ds(base + cc * CHUNK, CHUNK)])

    issue(0, 0)

    @pl.loop(0, NCHUNK - 1, step=2)
    def _chunk_pair(c):
        for k in range(2):
            cc = c + k
            issue(cc + 1, 1 - k)
            wait(cc, k)
            compute(cc, k)

    wait(NCHUNK - 1, 0)
    compute(NCHUNK - 1, 0)


def _edge_mask(U, V, row2d, col2d, w2, b2vec):
    mesh = plsc.VectorSubcoreMesh(core_axis_name="c", subcore_axis_name="s")
    run = functools.partial(
        pl.kernel,
        mesh=mesh,
        out_type=jax.ShapeDtypeStruct((N_EDGES,), jnp.float32),
        compiler_params=pltpu.CompilerParams(needs_layout_passes=False),
        scratch_types=[
            pltpu.VMEM((NCHUNK, CHUNK), jnp.int32),
            pltpu.VMEM((NCHUNK, CHUNK), jnp.int32),
            pltpu.VMEM((2, CHUNK, DIM), jnp.float32),
            pltpu.VMEM((2, CHUNK, DIM), jnp.float32),
            pltpu.VMEM((CHUNK,), jnp.float32),
            pltpu.VMEM((DIM,), jnp.float32),
            pltpu.VMEM((L,), jnp.float32),
            pltpu.SemaphoreType.DMA((2,)),
        ],
    )(_edge_body)
    return run(U, V, row2d, col2d, w2, b2vec)


def kernel(x, edge_index, W1, b1, W2, b2):
    row = edge_index[0].astype(jnp.int32).reshape(NW, NCHUNK, CHUNK)
    col = edge_index[1].astype(jnp.int32).reshape(NW, NCHUNK, CHUNK)
    U, V = _node_projections(x, W1, b1)
    w2 = W2.reshape(DIM)
    b2vec = jnp.broadcast_to(b2, (L,)).astype(jnp.float32)
    return _edge_mask(U, V, row, col, w2, b2vec)


# speedup vs baseline: 7.9246x; 1.6616x over previous
import functools

import jax
import jax.numpy as jnp
from jax import lax
from jax.experimental import pallas as pl
from jax.experimental.pallas import tpu as pltpu
from jax.experimental.pallas import tpu_sc as plsc

N_NODES = 10000
N_EDGES = 320000
DIM = 128
NC = 2
NS = 16
NW = NC * NS
L = 16
EPW = N_EDGES // NW
CHUNK = 80
NCHUNK = EPW // CHUNK
NF = DIM // L


def _proj_body(x_ref, wa_ref, wb_ref, b1_ref, u_ref, v_ref):
    xv = x_ref[...]
    u_ref[...] = jnp.dot(xv, wa_ref[...], preferred_element_type=jnp.float32)
    v_ref[...] = (
        jnp.dot(xv, wb_ref[...], preferred_element_type=jnp.float32) + b1_ref[...]
    )


def _node_projections(x, W1, b1):
    wa = W1[:, :DIM].T
    wb = W1[:, DIM:].T
    return pl.pallas_call(
        _proj_body,
        out_shape=[
            jax.ShapeDtypeStruct((N_NODES, DIM), jnp.float32),
            jax.ShapeDtypeStruct((N_NODES, DIM), jnp.float32),
        ],
    )(x, wa, wb, b1.reshape(1, DIM))


def _edge_body(u_hbm, v_hbm, row_hbm, col_hbm, w2_hbm, b2_hbm, out_hbm,
               idx_r, idx_c, urows, vrows, logv, w2v, b2v, sem):
    cid = lax.axis_index("c")
    sid = lax.axis_index("s")
    wid = sid * NC + cid
    base = wid * EPW

    pltpu.sync_copy(row_hbm.at[wid], idx_r)
    pltpu.sync_copy(col_hbm.at[wid], idx_c)
    pltpu.sync_copy(w2_hbm, w2v)
    pltpu.sync_copy(b2_hbm, b2v)
    b2reg = b2v[...]
    w2regs = [w2v[pl.ds(b * L, L)] for b in range(NF)]
    lane = lax.iota(jnp.int32, L)

    def issue(cc, s):
        pltpu.async_copy(u_hbm.at[idx_r.at[cc]], urows.at[s], sem.at[s])
        pltpu.async_copy(v_hbm.at[idx_c.at[cc]], vrows.at[s], sem.at[s])

    def wait(cc, s):
        pltpu.make_async_copy(u_hbm.at[idx_r.at[cc]], urows.at[s], sem.at[s]).wait()
        pltpu.make_async_copy(v_hbm.at[idx_c.at[cc]], vrows.at[s], sem.at[s]).wait()

    def compute(cc, s):
        ur = urows.at[s]
        vr = vrows.at[s]

        def group_body(g, carry):
            def edge_body(j, merged):
                e = g * L + j
                acc = jnp.maximum(ur[e, pl.ds(0, L)] + vr[e, pl.ds(0, L)], 0.0)
                acc = acc * w2regs[0]
                for b in range(1, NF):
                    h = jnp.maximum(
                        ur[e, pl.ds(b * L, L)] + vr[e, pl.ds(b * L, L)], 0.0
                    )
                    acc = acc + h * w2regs[b]
                return jnp.where(lane == j, jnp.sum(acc), merged)

            merged = lax.fori_loop(0, L, edge_body,
                                   jnp.zeros((L,), jnp.float32), unroll=4)
            z = merged + b2reg
            logv[pl.ds(g * L, L)] = 1.0 / (1.0 + jnp.exp(-z))
            return carry

        lax.fori_loop(0, CHUNK // L, group_body, 0, unroll=False)
        pltpu.sync_copy(logv, out_hbm.at[pl.ds(base + cc * CHUNK, CHUNK)])

    issue(0, 0)

    @pl.loop(0, NCHUNK - 1, step=2)
    def _chunk_pair(c):
        for k in range(2):
            cc = c + k
            issue(cc + 1, 1 - k)
            wait(cc, k)
            compute(cc, k)

    wait(NCHUNK - 1, 0)
    compute(NCHUNK - 1, 0)


def _edge_mask(U, V, row2d, col2d, w2, b2vec):
    mesh = plsc.VectorSubcoreMesh(core_axis_name="c", subcore_axis_name="s")
    run = functools.partial(
        pl.kernel,
        mesh=mesh,
        out_type=jax.ShapeDtypeStruct((N_EDGES,), jnp.float32),
        compiler_params=pltpu.CompilerParams(needs_layout_passes=False),
        scratch_types=[
            pltpu.VMEM((NCHUNK, CHUNK), jnp.int32),
            pltpu.VMEM((NCHUNK, CHUNK), jnp.int32),
            pltpu.VMEM((2, CHUNK, DIM), jnp.float32),
            pltpu.VMEM((2, CHUNK, DIM), jnp.float32),
            pltpu.VMEM((CHUNK,), jnp.float32),
            pltpu.VMEM((DIM,), jnp.float32),
            pltpu.VMEM((L,), jnp.float32),
            pltpu.SemaphoreType.DMA((2,)),
        ],
    )(_edge_body)
    return run(U, V, row2d, col2d, w2, b2vec)


def kernel(x, edge_index, W1, b1, W2, b2):
    row = edge_index[0].astype(jnp.int32).reshape(NW, NCHUNK, CHUNK)
    col = edge_index[1].astype(jnp.int32).reshape(NW, NCHUNK, CHUNK)
    U, V = _node_projections(x, W1, b1)
    w2 = W2.reshape(DIM)
    b2vec = jnp.broadcast_to(b2, (L,)).astype(jnp.float32)
    return _edge_mask(U, V, row, col, w2, b2vec)
